# Initial kernel scaffold; baseline (speedup 1.0000x reference)
import jax
import jax.numpy as jnp
from jax.experimental import pallas as pl


def kernel(node_feats, edge_index, edge_feats, demand_pairs, demand_feats, W_node, b_node, W_edge, b_edge, msg_W0, msg_b0, upd_W0, upd_b0, msg_W1, msg_b1, upd_W1, upd_b1, ro_W1, ro_b1, ro_W2, ro_b2):
    raise NotImplementedError("write your pallas kernel here")


# speedup vs baseline: 3.8708x; 3.8708x over previous
import functools

import jax
import jax.numpy as jnp
from jax import lax
from jax.experimental import pallas as pl
from jax.experimental.pallas import tpu as pltpu
from jax.experimental.pallas import tpu_sc as plsc

NC, NS, L = 2, 16, 16
H = 64
CHUNK = 200


def _node_prep_body(nf, wn, bn, mws, mwd, h_out, gs_out, gd_out):
    h = jnp.maximum(jnp.dot(nf[...], wn[...],
                            preferred_element_type=jnp.float32) + bn[...], 0.0)
    h_out[...] = h
    gs_out[...] = jnp.dot(h, mws[...], preferred_element_type=jnp.float32)
    gd_out[...] = jnp.dot(h, mwd[...], preferred_element_type=jnp.float32)


def _node_prep(nf, wn, bn, mws, mwd):
    n = nf.shape[0]
    return pl.pallas_call(
        _node_prep_body,
        out_shape=[jax.ShapeDtypeStruct((n, H), jnp.float32)] * 3,
    )(nf, wn, bn, mws, mwd)


def _edge_prep_body(ef, we, be, m0e, b0, m1e, b1, hp0_out, hp1_out):
    he = jnp.maximum(jnp.dot(ef[...], we[...],
                             preferred_element_type=jnp.float32) + be[...], 0.0)
    hp0_out[...] = jnp.dot(he, m0e[...], preferred_element_type=jnp.float32) + b0[...]
    hp1_out[...] = jnp.dot(he, m1e[...], preferred_element_type=jnp.float32) + b1[...]


def _edge_prep(ef, we, be, m0e, b0, m1e, b1):
    e = ef.shape[0]
    blk = 8000
    grid = e // blk
    return pl.pallas_call(
        _edge_prep_body,
        grid=(grid,),
        in_specs=[
            pl.BlockSpec((blk, ef.shape[1]), lambda i: (i, 0)),
            pl.BlockSpec(we.shape, lambda i: (0, 0)),
            pl.BlockSpec(be.shape, lambda i: (0, 0)),
            pl.BlockSpec(m0e.shape, lambda i: (0, 0)),
            pl.BlockSpec(b0.shape, lambda i: (0, 0)),
            pl.BlockSpec(m1e.shape, lambda i: (0, 0)),
            pl.BlockSpec(b1.shape, lambda i: (0, 0)),
        ],
        out_specs=[pl.BlockSpec((blk, H), lambda i: (i, 0))] * 2,
        out_shape=[jax.ShapeDtypeStruct((e, H), jnp.float32)] * 2,
    )(ef, we, be, m0e, b0, m1e, b1)


def _update_body(h, agg2, uwh, uwa, ub, nws, nwd, hn_out, gs_out, gd_out):
    agg = agg2[0] + agg2[1]
    hn = jnp.maximum(
        jnp.dot(h[...], uwh[...], preferred_element_type=jnp.float32)
        + jnp.dot(agg, uwa[...], preferred_element_type=jnp.float32)
        + ub[...], 0.0)
    hn_out[...] = hn
    gs_out[...] = jnp.dot(hn, nws[...], preferred_element_type=jnp.float32)
    gd_out[...] = jnp.dot(hn, nwd[...], preferred_element_type=jnp.float32)


def _update(h, agg2, uwh, uwa, ub, nws, nwd):
    n = h.shape[0]
    return pl.pallas_call(
        _update_body,
        out_shape=[jax.ShapeDtypeStruct((n, H), jnp.float32)] * 3,
    )(h, agg2, uwh, uwa, ub, nws, nwd)


def _demand_prep_body(df, w1f, b1, dfp_out):
    dfp_out[...] = jnp.dot(df[...], w1f[...],
                           preferred_element_type=jnp.float32) + b1[...]


def _demand_prep(df, w1f, b1):
    d = df.shape[0]
    blk = 10000
    return pl.pallas_call(
        _demand_prep_body,
        grid=(d // blk,),
        in_specs=[
            pl.BlockSpec((blk, df.shape[1]), lambda i: (i, 0)),
            pl.BlockSpec(w1f.shape, lambda i: (0, 0)),
            pl.BlockSpec(b1.shape, lambda i: (0, 0)),
        ],
        out_specs=pl.BlockSpec((blk, H), lambda i: (i, 0)),
        out_shape=jax.ShapeDtypeStruct((d, H), jnp.float32),
    )(df, w1f, b1)


def _final_body(t16, b2, out):
    out[...] = jax.nn.sigmoid(jnp.sum(t16[...], axis=1) + b2[0, 0])


def _final(t16, b2):
    d = t16.shape[0]
    return pl.pallas_call(
        _final_body,
        out_shape=jax.ShapeDtypeStruct((d,), jnp.float32),
    )(t16, b2)


def _sc_round(src, dst, hp, gs, gd):
    e = src.shape[0]
    n = gs.shape[0]
    per_w = e // (NC * NS)
    n_chunks = per_w // CHUNK
    zc = n // CHUNK
    mesh = plsc.VectorSubcoreMesh(core_axis_name="c", subcore_axis_name="s")

    @functools.partial(
        pl.kernel,
        out_type=jax.ShapeDtypeStruct((NC, n, H), jnp.float32),
        mesh=mesh,
        compiler_params=pltpu.CompilerParams(use_tc_tiling_on_sc=False),
        scratch_types=[
            pltpu.VMEM((CHUNK,), jnp.int32),
            pltpu.VMEM((CHUNK,), jnp.int32),
            pltpu.VMEM((CHUNK, H), jnp.float32),
            pltpu.VMEM((CHUNK, H), jnp.float32),
            pltpu.VMEM((CHUNK, H), jnp.float32),
            pltpu.VMEM_SHARED((n, H), jnp.float32),
            pltpu.SemaphoreType.DMA,
            pltpu.SemaphoreType.DMA,
        ],
    )
    def k(src_hbm, dst_hbm, hp_hbm, gs_hbm, gd_hbm, out_hbm,
          src_v, dst_v, rows_s, rows_d, msg_v, shared, sem1, sem2):
        c = lax.axis_index("c")
        s = lax.axis_index("s")

        def zbody(i, _):
            for j in range(H // L):
                msg_v[i, pl.ds(j * L, L)] = jnp.zeros((L,), jnp.float32)
            return 0
        lax.fori_loop(0, CHUNK, zbody, 0)
        for t in range((zc + NS - 1) // NS):
            kk = s + NS * t
            @pl.when(kk < zc)
            def _():
                pltpu.sync_copy(msg_v, shared.at[pl.ds(kk * CHUNK, CHUNK)])
        plsc.subcore_barrier()

        w = c * NS + s
        base_w = w * per_w

        def body(i, _):
            base = base_w + i * CHUNK
            pltpu.sync_copy(src_hbm.at[pl.ds(base, CHUNK)], src_v)
            pltpu.sync_copy(dst_hbm.at[pl.ds(base, CHUNK)], dst_v)
            cp1 = pltpu.async_copy(gs_hbm.at[src_v], rows_s, sem1)
            cp2 = pltpu.async_copy(gd_hbm.at[dst_v], rows_d, sem2)
            pltpu.sync_copy(hp_hbm.at[pl.ds(base, CHUNK)], msg_v)
            cp1.wait()
            cp2.wait()

            def crow(r, _):
                for j in range(H // L):
                    sl = pl.ds(j * L, L)
                    v = rows_s[r, sl] + rows_d[r, sl] + msg_v[r, sl]
                    msg_v[r, sl] = jnp.maximum(v, 0.0)
                return 0
            lax.fori_loop(0, CHUNK, crow, 0)
            pltpu.sync_copy(msg_v, shared.at[dst_v], add=True)
            return 0
        lax.fori_loop(0, n_chunks, body, 0)

        plsc.subcore_barrier()
        for t in range((zc + NS - 1) // NS):
            kk = s + NS * t
            @pl.when(kk < zc)
            def _():
                pltpu.sync_copy(shared.at[pl.ds(kk * CHUNK, CHUNK)],
                                out_hbm.at[c, pl.ds(kk * CHUNK, CHUNK)])

    return k(src, dst, hp, gs, gd)


def _sc_readout(ds, dd, a_tab, b_tab, dfp, w2):
    d = ds.shape[0]
    total_chunks = d // CHUNK
    per_w_max = (total_chunks + NC * NS - 1) // (NC * NS)
    mesh = plsc.VectorSubcoreMesh(core_axis_name="c", subcore_axis_name="s")

    @functools.partial(
        pl.kernel,
        out_type=jax.ShapeDtypeStruct((d, L), jnp.float32),
        mesh=mesh,
        compiler_params=pltpu.CompilerParams(use_tc_tiling_on_sc=False),
        scratch_types=[
            pltpu.VMEM((CHUNK,), jnp.int32),
            pltpu.VMEM((CHUNK,), jnp.int32),
            pltpu.VMEM((CHUNK, H), jnp.float32),
            pltpu.VMEM((CHUNK, H), jnp.float32),
            pltpu.VMEM((CHUNK, H), jnp.float32),
            pltpu.VMEM((CHUNK, L), jnp.float32),
            pltpu.VMEM((H,), jnp.float32),
            pltpu.SemaphoreType.DMA,
            pltpu.SemaphoreType.DMA,
        ],
    )
    def k(ds_hbm, dd_hbm, a_hbm, b_hbm, dfp_hbm, w2_hbm, out_hbm,
          ds_v, dd_v, rows_a, rows_b, dfp_v, out16_v, w2_v, sem1, sem2):
        c = lax.axis_index("c")
        s = lax.axis_index("s")
        w = c * NS + s
        pltpu.sync_copy(w2_hbm, w2_v)

        def body(i, _):
            kk = w + (NC * NS) * i
            @pl.when(kk < total_chunks)
            def _():
                base = kk * CHUNK
                pltpu.sync_copy(ds_hbm.at[pl.ds(base, CHUNK)], ds_v)
                pltpu.sync_copy(dd_hbm.at[pl.ds(base, CHUNK)], dd_v)
                cp1 = pltpu.async_copy(a_hbm.at[ds_v], rows_a, sem1)
                cp2 = pltpu.async_copy(b_hbm.at[dd_v], rows_b, sem2)
                pltpu.sync_copy(dfp_hbm.at[pl.ds(base, CHUNK)], dfp_v)
                cp1.wait()
                cp2.wait()

                def crow(r, _):
                    t = jnp.zeros((L,), jnp.float32)
                    for j in range(H // L):
                        sl = pl.ds(j * L, L)
                        x = jnp.maximum(
                            rows_a[r, sl] + rows_b[r, sl] + dfp_v[r, sl], 0.0)
                        t = t + x * w2_v[sl]
                    out16_v[r, :] = t
                    return 0
                lax.fori_loop(0, CHUNK, crow, 0)
                pltpu.sync_copy(out16_v, out_hbm.at[pl.ds(base, CHUNK)])
            return 0
        lax.fori_loop(0, per_w_max, body, 0)

    return k(ds, dd, a_tab, b_tab, dfp, w2)


def kernel(node_feats, edge_index, edge_feats, demand_pairs, demand_feats,
           W_node, b_node, W_edge, b_edge,
           msg_W0, msg_b0, upd_W0, upd_b0,
           msg_W1, msg_b1, upd_W1, upd_b1,
           ro_W1, ro_b1, ro_W2, ro_b2):
    f32 = jnp.float32
    src = edge_index[0].astype(jnp.int32)
    dst = edge_index[1].astype(jnp.int32)
    ds = demand_pairs[:, 0].astype(jnp.int32)
    dd = demand_pairs[:, 1].astype(jnp.int32)

    bn = b_node.astype(f32).reshape(1, H)
    be = b_edge.astype(f32).reshape(1, H)
    mb0 = msg_b0.astype(f32).reshape(1, H)
    mb1 = msg_b1.astype(f32).reshape(1, H)
    ub0 = upd_b0.astype(f32).reshape(1, H)
    ub1 = upd_b1.astype(f32).reshape(1, H)
    rb1 = ro_b1.astype(f32).reshape(1, H)
    rb2 = ro_b2.astype(f32).reshape(1, 1)
    w2 = ro_W2.astype(f32).reshape(H)

    hp0, hp1 = _edge_prep(edge_feats, W_edge, be,
                          msg_W0[H:2 * H], mb0, msg_W1[H:2 * H], mb1)
    dfp = _demand_prep(demand_feats, ro_W1[2 * H:], rb1)

    h, gs, gd = _node_prep(node_feats, W_node, bn,
                           msg_W0[:H], msg_W0[2 * H:])

    agg2 = _sc_round(src, dst, hp0, gs, gd)
    h, gs, gd = _update(h, agg2, upd_W0[:H], upd_W0[H:], ub0,
                        msg_W1[:H], msg_W1[2 * H:])

    agg2 = _sc_round(src, dst, hp1, gs, gd)
    h, a_tab, b_tab = _update(h, agg2, upd_W1[:H], upd_W1[H:], ub1,
                              ro_W1[:H], ro_W1[H:2 * H])

    t16 = _sc_readout(ds, dd, a_tab, b_tab, dfp, w2)
    return _final(t16, rb2)
